# Initial kernel scaffold; baseline (speedup 1.0000x reference)
import jax
import jax.numpy as jnp
from jax.experimental import pallas as pl


def kernel(x, table):
    raise NotImplementedError("write your pallas kernel here")


# speedup vs baseline: 1.6997x; 1.6997x over previous
import jax
import jax.numpy as jnp
from jax.experimental import pallas as pl

BL = 256


def _add_kernel(x_ref, t_ref, o_ref):
    o_ref[...] = x_ref[...] + t_ref[...]


def kernel(x, table):
    B, L, D = x.shape
    grid = (L // BL, B)
    return pl.pallas_call(
        _add_kernel,
        grid=grid,
        in_specs=[
            pl.BlockSpec((1, BL, D), lambda j, b: (b, j, 0)),
            pl.BlockSpec((BL, D), lambda j, b: (j, 0)),
        ],
        out_specs=pl.BlockSpec((1, BL, D), lambda j, b: (b, j, 0)),
        out_shape=jax.ShapeDtypeStruct((B, L, D), x.dtype),
    )(x, table)
